# speedup vs baseline: 3.3273x; 3.3273x over previous
import jax
import jax.numpy as jnp
from jax.experimental import pallas as pl

_D = 768
_PE_PAD = 128
_B = 1024


def _body(idx_ref, x_ref, pe_ref, o_ref):
    idx = idx_ref[0, 0, :]
    iota = jax.lax.broadcasted_iota(jnp.int32, (_B, _PE_PAD), 1)
    onehot = jnp.where(idx[:, None] == iota, 1.0, 0.0).astype(jnp.float32)
    o_ref[...] = x_ref[...] + jax.lax.dot_general(
        onehot, pe_ref[...], (((1,), (0,)), ((), ())),
        preferred_element_type=jnp.float32)


def kernel(x, layer_index, pe):
    n = x.shape[0]
    grid = n // _B
    pe2 = jnp.zeros((_PE_PAD, _D), jnp.float32).at[: pe.shape[0]].set(
        pe.reshape(pe.shape[0], _D))
    idx3 = layer_index.reshape(grid, 1, _B)
    return pl.pallas_call(
        _body,
        grid=(grid,),
        in_specs=[
            pl.BlockSpec((1, 1, _B), lambda i: (i, 0, 0)),
            pl.BlockSpec((_B, _D), lambda i: (i, 0)),
            pl.BlockSpec((_PE_PAD, _D), lambda i: (0, 0)),
        ],
        out_specs=pl.BlockSpec((_B, _D), lambda i: (i, 0)),
        out_shape=jax.ShapeDtypeStruct((n, _D), jnp.float32),
    )(idx3, x, pe2)
